# baseline (speedup 1.0000x reference)
import functools

import jax
import jax.numpy as jnp
from jax import lax
from jax.experimental import pallas as pl
from jax.experimental.pallas import tpu as pltpu
from jax.experimental.pallas import tpu_sc as plsc

B = 16384
D = 64
FD = 128
NC = 2
NS = 16
NW = NC * NS
BPW = B // NW
C = 256
NCH = BPW // C


def _sc_body(ut_hbm, mt_hbm, x_hbm, idxm_hbm, idxu_hbm, wb_hbm, out_hbm,
             idxm_v, idxu_v, wb_v, urows, mrows, xrows, out_v, pp_v,
             sem_u, sem_m, sem_x):
    wid = lax.axis_index("s") * NC + lax.axis_index("c")
    base = wid * BPW
    pltpu.sync_copy(idxm_hbm.at[pl.ds(base, BPW)], idxm_v)
    pltpu.sync_copy(idxu_hbm.at[pl.ds(base, BPW)], idxu_v)
    pltpu.sync_copy(wb_hbm, wb_v)

    w_u = [wb_v[pl.ds(16 * k, 16)] for k in range(D // 16)]
    w_m = [wb_v[pl.ds(D + 16 * k, 16)] for k in range(D // 16)]
    w_x = [wb_v[pl.ds(2 * D + 16 * k, 16)] for k in range(FD // 16)]
    bias = wb_v[pl.ds(2 * D + FD, 16)]
    rowoff = lax.iota(jnp.int32, 16) * 16

    for c in range(NCH):
        cbase = c * C
        cp_u = pltpu.async_copy(ut_hbm.at[idxu_v.at[pl.ds(cbase, C)]], urows, sem_u)
        cp_m = pltpu.async_copy(mt_hbm.at[idxm_v.at[pl.ds(cbase, C)]], mrows, sem_m)
        cp_x = pltpu.async_copy(x_hbm.at[pl.ds(base + cbase, C)], xrows, sem_x)
        cp_u.wait()
        cp_m.wait()
        cp_x.wait()

        def grp_body(g, carry, cbase=cbase):
            for j in range(16):
                i = g * 16 + j
                acc = bias
                for k in range(D // 16):
                    acc = acc + urows[i, pl.ds(16 * k, 16)] * w_u[k]
                for k in range(D // 16):
                    acc = acc + mrows[i, pl.ds(16 * k, 16)] * w_m[k]
                for k in range(FD // 16):
                    acc = acc + xrows[i, pl.ds(16 * k, 16)] * w_x[k]
                pp_v[pl.ds(16 * j, 16)] = acc
            tot = jnp.zeros((16,), jnp.float32)
            for j in range(16):
                tot = tot + plsc.load_gather(pp_v, [rowoff + j])
            out_v[pl.ds(cbase + g * 16, 16)] = tot
            return carry

        lax.fori_loop(0, C // 16, grp_body, 0)

    pltpu.sync_copy(out_v, out_hbm.at[pl.ds(base, BPW)])


@functools.partial(jax.jit, static_argnums=())
def _sc_call(ut, mt, x, idxm, idxu, wb):
    mesh = plsc.VectorSubcoreMesh(core_axis_name="c", subcore_axis_name="s")
    fn = functools.partial(
        pl.kernel,
        out_type=jax.ShapeDtypeStruct((B,), jnp.float32),
        mesh=mesh,
        compiler_params=pltpu.CompilerParams(
            needs_layout_passes=False, use_tc_tiling_on_sc=False),
        scratch_types=[
            pltpu.VMEM((BPW,), jnp.int32),
            pltpu.VMEM((BPW,), jnp.int32),
            pltpu.VMEM((2 * D + FD + 16,), jnp.float32),
            pltpu.VMEM((C, D), jnp.float32),
            pltpu.VMEM((C, D), jnp.float32),
            pltpu.VMEM((C, FD), jnp.float32),
            pltpu.VMEM((BPW,), jnp.float32),
            pltpu.VMEM((256,), jnp.float32),
            pltpu.SemaphoreType.DMA,
            pltpu.SemaphoreType.DMA,
            pltpu.SemaphoreType.DMA,
        ],
    )(_sc_body)
    return fn(ut, mt, x, idxm, idxu, wb)


def kernel(x, idxs, user_table, movie_table, W, b):
    idx32 = idxs.astype(jnp.int32)
    idxm = idx32[:, 0]
    idxu = idx32[:, 1]
    wb = jnp.concatenate(
        [W[:, 0], jnp.broadcast_to(b.astype(jnp.float32) / 16.0, (16,))])
    out = _sc_call(user_table, movie_table, x, idxm, idxu, wb)
    return out.reshape(B, 1)


# speedup vs baseline: 5.7168x; 5.7168x over previous
import functools

import jax
import jax.numpy as jnp
from jax import lax
from jax.experimental import pallas as pl
from jax.experimental.pallas import tpu as pltpu
from jax.experimental.pallas import tpu_sc as plsc

B = 16384
N = 1000000
D = 64
FD = 128
NC = 2
NS = 16
NW = NC * NS
BPW = B // NW

GRID = 62
TBLK = 16384
NXB = 32
XBLK = B // NXB


def _tc_body(wb_ref, ttu_ref, ttm_ref, x_ref, vu_ref, vm_ref, xw_ref):
    wu = wb_ref[0:D][:, None]
    wm = wb_ref[D:2 * D][:, None]
    vu_ref[...] = jnp.sum(ttu_ref[...] * wu, axis=0)
    vm_ref[...] = jnp.sum(ttm_ref[...] * wm, axis=0)
    wx = wb_ref[2 * D:2 * D + FD][:, None]
    bias = wb_ref[2 * D + FD]
    xw_ref[...] = jnp.dot(x_ref[...], wx, preferred_element_type=jnp.float32)[:, 0] + bias


def _tc_call(ttu, ttm, x, wb):
    return pl.pallas_call(
        _tc_body,
        grid=(GRID,),
        in_specs=[
            pl.BlockSpec((2 * D + FD + 16,), lambda i: (0,)),
            pl.BlockSpec((D, TBLK), lambda i: (0, i)),
            pl.BlockSpec((D, TBLK), lambda i: (0, i)),
            pl.BlockSpec((XBLK, FD), lambda i: (i % NXB, 0)),
        ],
        out_specs=[
            pl.BlockSpec((TBLK,), lambda i: (i,)),
            pl.BlockSpec((TBLK,), lambda i: (i,)),
            pl.BlockSpec((XBLK,), lambda i: (i % NXB,)),
        ],
        out_shape=[
            jax.ShapeDtypeStruct((N,), jnp.float32),
            jax.ShapeDtypeStruct((N,), jnp.float32),
            jax.ShapeDtypeStruct((B,), jnp.float32),
        ],
    )(wb, ttu, ttm, x)


def _sc_body(vu_hbm, vm_hbm, xw_hbm, idxm_hbm, idxu_hbm, out_hbm,
             idxm_v, idxu_v, idxmq_v, idxuq_v, gu_v, gm_v, xw_v, out_v,
             sem_u, sem_m):
    wid = lax.axis_index("s") * NC + lax.axis_index("c")
    base = wid * BPW
    pltpu.sync_copy(idxm_hbm.at[pl.ds(base, BPW)], idxm_v)
    pltpu.sync_copy(idxu_hbm.at[pl.ds(base, BPW)], idxu_v)
    for k in range(BPW // 16):
        s = pl.ds(16 * k, 16)
        idxmq_v[s] = lax.shift_right_logical(idxm_v[s], 4)
        idxuq_v[s] = lax.shift_right_logical(idxu_v[s], 4)
    cp_u = pltpu.async_copy(vu_hbm.at[idxuq_v], gu_v, sem_u)
    cp_m = pltpu.async_copy(vm_hbm.at[idxmq_v], gm_v, sem_m)
    pltpu.sync_copy(xw_hbm.at[pl.ds(base, BPW)], xw_v)
    cp_u.wait()
    cp_m.wait()
    rows16 = lax.iota(jnp.int32, 16)
    for k in range(BPW // 16):
        s = pl.ds(16 * k, 16)
        rows = rows16 + 16 * k
        su = plsc.load_gather(gu_v, [rows, idxu_v[s] & 15])
        sm = plsc.load_gather(gm_v, [rows, idxm_v[s] & 15])
        out_v[s] = su + sm + xw_v[s]
    pltpu.sync_copy(out_v, out_hbm.at[pl.ds(base, BPW)])


def _sc_call(vu2, vm2, xw, idxm, idxu):
    mesh = plsc.VectorSubcoreMesh(core_axis_name="c", subcore_axis_name="s")
    fn = functools.partial(
        pl.kernel,
        out_type=jax.ShapeDtypeStruct((B,), jnp.float32),
        mesh=mesh,
        compiler_params=pltpu.CompilerParams(
            needs_layout_passes=False, use_tc_tiling_on_sc=False),
        scratch_types=[
            pltpu.VMEM((BPW,), jnp.int32),
            pltpu.VMEM((BPW,), jnp.int32),
            pltpu.VMEM((BPW,), jnp.int32),
            pltpu.VMEM((BPW,), jnp.int32),
            pltpu.VMEM((BPW, 16), jnp.float32),
            pltpu.VMEM((BPW, 16), jnp.float32),
            pltpu.VMEM((BPW,), jnp.float32),
            pltpu.VMEM((BPW,), jnp.float32),
            pltpu.SemaphoreType.DMA,
            pltpu.SemaphoreType.DMA,
        ],
    )(_sc_body)
    return fn(vu2, vm2, xw, idxm, idxu)


def kernel(x, idxs, user_table, movie_table, W, b):
    idx32 = idxs.astype(jnp.int32)
    idxm = idx32[:, 0]
    idxu = idx32[:, 1]
    wb = jnp.concatenate(
        [W[:, 0], jnp.broadcast_to(b.astype(jnp.float32), (16,))])
    vu, vm, xw = _tc_call(user_table.T, movie_table.T, x, wb)
    out = _sc_call(vu.reshape(N // 16, 16), vm.reshape(N // 16, 16),
                   xw, idxm, idxu)
    return out.reshape(B, 1)


# speedup vs baseline: 6.2562x; 1.0944x over previous
import functools

import jax
import jax.numpy as jnp
from jax import lax
from jax.experimental import pallas as pl
from jax.experimental.pallas import tpu as pltpu
from jax.experimental.pallas import tpu_sc as plsc

B = 16384
N = 1000000
D = 64
FD = 128
NC = 2
NS = 16
NW = NC * NS
BPW = B // NW

GRID = 62
TBLK = 16384
NXB = 32
XBLK = B // NXB


def _tc_body(wb_ref, ttu_ref, ttm_ref, x_ref, vu_ref, vm_ref, xw_ref):
    wu = wb_ref[0:D][None, :]
    wm = wb_ref[D:2 * D][None, :]
    dot = functools.partial(
        jax.lax.dot_general,
        dimension_numbers=(((1,), (0,)), ((), ())),
        preferred_element_type=jnp.float32,
    )
    vu_ref[...] = dot(wu, ttu_ref[...])[None]
    vm_ref[...] = dot(wm, ttm_ref[...])[None]
    wx = wb_ref[2 * D:2 * D + FD][:, None]
    bias = wb_ref[2 * D + FD]
    xw_ref[...] = jnp.dot(x_ref[...], wx, preferred_element_type=jnp.float32)[:, 0] + bias


def _tc_call(ttu, ttm, x, wb):
    return pl.pallas_call(
        _tc_body,
        grid=(GRID,),
        in_specs=[
            pl.BlockSpec((2 * D + FD + 16,), lambda i: (0,)),
            pl.BlockSpec((D, TBLK), lambda i: (0, i)),
            pl.BlockSpec((D, TBLK), lambda i: (0, i)),
            pl.BlockSpec((XBLK, FD), lambda i: (i % NXB, 0)),
        ],
        out_specs=[
            pl.BlockSpec((1, 1, TBLK), lambda i: (i, 0, 0)),
            pl.BlockSpec((1, 1, TBLK), lambda i: (i, 0, 0)),
            pl.BlockSpec((XBLK,), lambda i: (i % NXB,)),
        ],
        out_shape=[
            jax.ShapeDtypeStruct((GRID, 1, TBLK), jnp.float32),
            jax.ShapeDtypeStruct((GRID, 1, TBLK), jnp.float32),
            jax.ShapeDtypeStruct((B,), jnp.float32),
        ],
    )(wb, ttu, ttm, x)


def _sc_body(vu_hbm, vm_hbm, xw_hbm, idxm_hbm, idxu_hbm, out_hbm,
             idxm_v, idxu_v, idxmq_v, idxuq_v, gu_v, gm_v, xw_v, out_v,
             sem_u, sem_m):
    wid = lax.axis_index("s") * NC + lax.axis_index("c")
    base = wid * BPW
    pltpu.sync_copy(idxm_hbm.at[pl.ds(base, BPW)], idxm_v)
    pltpu.sync_copy(idxu_hbm.at[pl.ds(base, BPW)], idxu_v)
    for k in range(BPW // 16):
        s = pl.ds(16 * k, 16)
        idxmq_v[s] = lax.shift_right_logical(idxm_v[s], 4)
        idxuq_v[s] = lax.shift_right_logical(idxu_v[s], 4)
    cp_u = pltpu.async_copy(vu_hbm.at[idxuq_v], gu_v, sem_u)
    cp_m = pltpu.async_copy(vm_hbm.at[idxmq_v], gm_v, sem_m)
    pltpu.sync_copy(xw_hbm.at[pl.ds(base, BPW)], xw_v)
    cp_u.wait()
    cp_m.wait()
    rows16 = lax.iota(jnp.int32, 16)
    for k in range(BPW // 16):
        s = pl.ds(16 * k, 16)
        rows = rows16 + 16 * k
        su = plsc.load_gather(gu_v, [rows, idxu_v[s] & 15])
        sm = plsc.load_gather(gm_v, [rows, idxm_v[s] & 15])
        out_v[s] = su + sm + xw_v[s]
    pltpu.sync_copy(out_v, out_hbm.at[pl.ds(base, BPW)])


def _sc_call(vu2, vm2, xw, idxm, idxu):
    mesh = plsc.VectorSubcoreMesh(core_axis_name="c", subcore_axis_name="s")
    fn = functools.partial(
        pl.kernel,
        out_type=jax.ShapeDtypeStruct((B,), jnp.float32),
        mesh=mesh,
        compiler_params=pltpu.CompilerParams(
            needs_layout_passes=False, use_tc_tiling_on_sc=False),
        scratch_types=[
            pltpu.VMEM((BPW,), jnp.int32),
            pltpu.VMEM((BPW,), jnp.int32),
            pltpu.VMEM((BPW,), jnp.int32),
            pltpu.VMEM((BPW,), jnp.int32),
            pltpu.VMEM((BPW, 16), jnp.float32),
            pltpu.VMEM((BPW, 16), jnp.float32),
            pltpu.VMEM((BPW,), jnp.float32),
            pltpu.VMEM((BPW,), jnp.float32),
            pltpu.SemaphoreType.DMA,
            pltpu.SemaphoreType.DMA,
        ],
    )(_sc_body)
    return fn(vu2, vm2, xw, idxm, idxu)


def kernel(x, idxs, user_table, movie_table, W, b):
    idx32 = idxs.astype(jnp.int32)
    idxm = idx32[:, 0]
    idxu = idx32[:, 1]
    wb = jnp.concatenate(
        [W[:, 0], jnp.broadcast_to(b.astype(jnp.float32), (16,))])
    vu, vm, xw = _tc_call(user_table.T, movie_table.T, x, wb)
    out = _sc_call(vu.reshape(GRID * TBLK // 16, 16),
                   vm.reshape(GRID * TBLK // 16, 16),
                   xw, idxm, idxu)
    return out.reshape(B, 1)
